# baseline (speedup 1.0000x reference)
import jax
import jax.numpy as jnp
from jax import lax
from jax.experimental import pallas as pl
from jax.experimental.pallas import tpu as pltpu
from jax.experimental.pallas import tpu_sc as plsc

NL = 100000
CD = 64
NB = 16384
LRC = 0.05
THETA = 0.3

CH = 16384
NTRASH = 128
NFULL = NL // CH
LAST = NL - NFULL * CH
EV_T = NB // 16
EV_W = NB // 32

_mesh = plsc.VectorSubcoreMesh(core_axis_name="c", subcore_axis_name="s")
_params = pltpu.CompilerParams(use_tc_tiling_on_sc=False)


def _sc_act_body(w0_hbm, idx_hbm, con_hbm, g_hbm,
                 conB, idxS, abuf, wbuf, sidxA, gidx, widx, sidxG, acc, sem):
    c = lax.axis_index("c")
    s = lax.axis_index("s")
    ev0 = s * EV_T

    if True:
        pltpu.sync_copy(idx_hbm.at[pl.ds(ev0, EV_T)], idxS)

        lane = lax.iota(jnp.int32, 16)
        zeros16 = jnp.zeros((16,), jnp.float32)

        def bw(g, _):
            iv = idxS[pl.ds(c * EV_W + g * 16, 16)]
            widx[g // 8, pl.ds((g % 8) * 16, 16)] = iv
            return 0
        lax.fori_loop(0, EV_W // 16, bw, 0)

        def chunk_body(k, _):
            kbase = k * CH

            def za(i, _):
                for j in range(4):
                    abuf[i, pl.ds(j * 16, 16)] = zeros16
                return 0
            lax.fori_loop(0, 128, za, 0)

            def z2(i, _):
                pltpu.sync_copy(abuf, acc.at[pl.ds(s * EV_T + i * 128, 128)])
                return 0
            lax.fori_loop(0, EV_T // 128, z2, 0)
            plsc.subcore_barrier()

            def spass(p, _):
                pltpu.sync_copy(
                    con_hbm.at[pl.ds(ev0 + p * EV_W, EV_W)], conB)

                def bs(g, _):
                    iv = idxS[pl.ds(p * EV_W + g * 16, 16)]
                    loc = iv - kbase
                    inb = (loc >= 0) & (loc < CH)
                    li = jnp.where(inb, loc,
                                   CH + ((iv + lane) & (NTRASH - 1)))
                    sidxA[g // 8, pl.ds((g % 8) * 16, 16)] = li
                    return 0
                lax.fori_loop(0, EV_W // 16, bs, 0)

                def sc(gr, _):
                    pltpu.sync_copy(conB.at[pl.ds(gr * 128, 128)],
                                    acc.at[sidxA.at[gr]], add=True)
                    return 0
                lax.fori_loop(0, 4, sc, 0)
                return 0
            lax.fori_loop(0, 2, spass, 0)
            plsc.subcore_barrier()

            def bm(g, _):
                iv = idxS[pl.ds(c * EV_W + g * 16, 16)]
                loc = iv - kbase
                inb = (loc >= 0) & (loc < CH)
                gidx[g // 8, pl.ds((g % 8) * 16, 16)] = iv & (CH - 1)
                gpos = ev0 + c * EV_W + g * 16 + lane
                sidxG[g // 8, pl.ds((g % 8) * 16, 16)] = jnp.where(
                    inb, gpos, NB + ((iv + lane) & (NTRASH - 1)))
                return 0
            lax.fori_loop(0, EV_W // 16, bm, 0)

            def gblk(blk, _):
                pltpu.async_copy(acc.at[gidx.at[blk]], abuf, sem).wait()
                pltpu.async_copy(w0_hbm.at[widx.at[blk]], wbuf, sem).wait()

                def comb(e, _):
                    for j in range(4):
                        sl = pl.ds(j * 16, 16)
                        wbuf[e, sl] = wbuf[e, sl] + abuf[e, sl] * LRC
                    return 0
                lax.fori_loop(0, 128, comb, 0)
                pltpu.async_copy(wbuf, g_hbm.at[sidxG.at[blk]], sem).wait()
                return 0
            lax.fori_loop(0, 4, gblk, 0)
            plsc.subcore_barrier()
            return 0

        lax.fori_loop(0, NFULL + 1, chunk_body, 0)


_sc_act = pl.kernel(
    _sc_act_body,
    out_type=jax.ShapeDtypeStruct((NB + NTRASH, CD), jnp.float32),
    mesh=_mesh,
    compiler_params=_params,
    scratch_types=[
        pltpu.VMEM((EV_W, CD), jnp.float32),
        pltpu.VMEM((EV_T,), jnp.int32),
        pltpu.VMEM((128, CD), jnp.float32),
        pltpu.VMEM((128, CD), jnp.float32),
        pltpu.VMEM((4, 128), jnp.int32),
        pltpu.VMEM((4, 128), jnp.int32),
        pltpu.VMEM((4, 128), jnp.int32),
        pltpu.VMEM((4, 128), jnp.int32),
        pltpu.VMEM_SHARED((CH + NTRASH, CD), jnp.float32),
        pltpu.SemaphoreType.DMA,
    ],
)


def _tc_gate_body(g_ref, c_ref, p_ref, v_ref):
    g = g_ref[...]
    cc = c_ref[...]
    act = jnp.sum(g * cc, axis=-1, keepdims=True)
    scale = jnp.where(act > THETA, jnp.float32(LRC), jnp.float32(0.0))
    v_ref[...] = scale * p_ref[...]


def _tc_gate(g, concept, phon):
    return pl.pallas_call(
        _tc_gate_body,
        grid=(NB // 2048,),
        in_specs=[pl.BlockSpec((2048, CD), lambda i: (i, 0))] * 3,
        out_specs=pl.BlockSpec((2048, CD), lambda i: (i, 0)),
        out_shape=jax.ShapeDtypeStruct((NB, CD), jnp.float32),
    )(g, concept, phon)


def _sc_out_body(wlp_hbm, idx_hbm, v_hbm, out_hbm,
                 vB, idxS, sidx, dstage, dacc, sem):
    c = lax.axis_index("c")
    s = lax.axis_index("s")
    ev0 = s * EV_T

    if True:
        pltpu.sync_copy(idx_hbm.at[pl.ds(ev0, EV_T)], idxS)

        lane = lax.iota(jnp.int32, 16)

        def scatter_chunk(kbase):
            def spass(p, _):
                pltpu.sync_copy(v_hbm.at[pl.ds(ev0 + p * EV_W, EV_W)], vB)

                def bs(g, _):
                    iv = idxS[pl.ds(p * EV_W + g * 16, 16)]
                    loc = iv - kbase
                    inb = (loc >= 0) & (loc < CH)
                    li = jnp.where(inb, loc,
                                   CH + ((iv + lane) & (NTRASH - 1)))
                    sidx[g // 8, pl.ds((g % 8) * 16, 16)] = li
                    return 0
                lax.fori_loop(0, EV_W // 16, bs, 0)

                def sc(gr, _):
                    pltpu.sync_copy(vB.at[pl.ds(gr * 128, 128)],
                                    dacc.at[sidx.at[gr]], add=True)
                    return 0
                lax.fori_loop(0, 4, sc, 0)
                return 0
            lax.fori_loop(0, 2, spass, 0)

        def full_chunk(kk, _):
            k = 2 * kk + c
            kbase = k * CH

            def di(i, _):
                pltpu.sync_copy(
                    wlp_hbm.at[pl.ds(kbase + s * EV_T + i * 128, 128)],
                    dstage)
                pltpu.sync_copy(
                    dstage, dacc.at[pl.ds(s * EV_T + i * 128, 128)])
                return 0
            lax.fori_loop(0, 8, di, 0)
            plsc.subcore_barrier()
            scatter_chunk(kbase)
            plsc.subcore_barrier()

            def wo(i, _):
                pltpu.sync_copy(
                    dacc.at[pl.ds(s * EV_T + i * 128, 128)], dstage)
                pltpu.sync_copy(
                    dstage,
                    out_hbm.at[pl.ds(kbase + s * EV_T + i * 128, 128)])
                return 0
            lax.fori_loop(0, 8, wo, 0)
            plsc.subcore_barrier()
            return 0

        lax.fori_loop(0, 3, full_chunk, 0)

        @pl.when(c == 0)
        def _():
            kbase = NFULL * CH
            rows_t = LAST // 16
            pltpu.sync_copy(wlp_hbm.at[pl.ds(kbase + s * rows_t, rows_t)],
                            dstage.at[pl.ds(0, rows_t)])
            pltpu.sync_copy(dstage.at[pl.ds(0, rows_t)],
                            dacc.at[pl.ds(s * rows_t, rows_t)])
            plsc.subcore_barrier()
            scatter_chunk(kbase)
            plsc.subcore_barrier()
            pltpu.sync_copy(dacc.at[pl.ds(s * rows_t, rows_t)],
                            dstage.at[pl.ds(0, rows_t)])
            pltpu.sync_copy(dstage.at[pl.ds(0, rows_t)],
                            out_hbm.at[pl.ds(kbase + s * rows_t, rows_t)])


_sc_out = pl.kernel(
    _sc_out_body,
    out_type=jax.ShapeDtypeStruct((NL, CD), jnp.float32),
    mesh=_mesh,
    compiler_params=_params,
    scratch_types=[
        pltpu.VMEM((EV_W, CD), jnp.float32),
        pltpu.VMEM((EV_T,), jnp.int32),
        pltpu.VMEM((4, 128), jnp.int32),
        pltpu.VMEM((128, CD), jnp.float32),
        pltpu.VMEM_SHARED((CH + NTRASH, CD), jnp.float32),
        pltpu.SemaphoreType.DMA,
    ],
)


def kernel(W_C_to_L, W_L_to_P, idx, concept, phon):
    idx = idx.astype(jnp.int32)
    gpad = _sc_act(W_C_to_L, idx, concept)
    v = _tc_gate(gpad[:NB], concept, phon)
    return _sc_out(W_L_to_P, idx, v)


# speedup vs baseline: 1.2063x; 1.2063x over previous
import jax
import jax.numpy as jnp
from jax import lax
from jax.experimental import pallas as pl
from jax.experimental.pallas import tpu as pltpu
from jax.experimental.pallas import tpu_sc as plsc

NL = 100000
CD = 64
NB = 16384
LRC = 0.05
THETA = 0.3

CH = 16384
CHB = 14
NTRASH = 128
NCH = 7
NFULL = NL // CH
LAST = NL - NFULL * CH
EV_T = NB // 16
EV_W = NB // 32

_mesh = plsc.VectorSubcoreMesh(core_axis_name="c", subcore_axis_name="s")
_params = pltpu.CompilerParams(use_tc_tiling_on_sc=False,
                               needs_layout_passes=False)

_i32 = jnp.int32


def _sort_events(idx_hbm, ev0, idxS, idxF, evid, smem):
    pltpu.sync_copy(idx_hbm.at[pl.ds(ev0, EV_T)], idxS.at[pl.ds(0, EV_T)])
    lane = lax.iota(_i32, 16)

    def cntf(g, cnts):
        ck = jnp.right_shift(idxS[pl.ds(g * 16, 16)], CHB)
        return tuple(cnts[k] + jnp.sum(jnp.where(ck == k, 1, 0))
                     for k in range(NCH))
    cnts = lax.fori_loop(0, EV_T // 16, cntf, (_i32(0),) * NCH)

    running = _i32(0)
    for k in range(NCH):
        smem[k] = running
        smem[8 + k] = running
        running = running + cnts[k]

    lane0 = lane == 0

    def fill(e, _):
        iv = idxS[pl.ds(e, 16)][0]
        k = jnp.right_shift(iv, CHB)
        p = smem[8 + k]
        smem[8 + k] = p + 1
        r = jnp.full((16,), jnp.right_shift(p, 7), _i32)
        cc = jnp.full((16,), p & 127, _i32)
        plsc.store_scatter(idxF, [r, cc], jnp.full((16,), iv, _i32),
                           mask=lane0)
        plsc.store_scatter(evid, [r, cc], jnp.full((16,), ev0 + e, _i32),
                           mask=lane0)
        return 0
    lax.fori_loop(0, EV_T, fill, 0)


def _sc_act_body(w0_hbm, idx_hbm, con_hbm, g_hbm,
                 idxS, idxF, evid, conB, abuf, wbuf, zbuf,
                 sidxB, gidxB, widxB, goutB, smem, acc, sem, sem2):
    c = lax.axis_index("c")
    s = lax.axis_index("s")
    ev0 = s * EV_T
    lane = lax.iota(_i32, 16)

    _sort_events(idx_hbm, ev0, idxS, idxF, evid, smem)

    zeros16 = jnp.zeros((16,), jnp.float32)

    def zb(i, _):
        for j in range(4):
            zbuf[i, pl.ds(j * 16, 16)] = zeros16
        return 0
    lax.fori_loop(0, 128, zb, 0)

    def z2(i, _):
        pltpu.sync_copy(zbuf, acc.at[pl.ds(s * EV_T + i * 128, 128)])
        return 0
    lax.fori_loop(0, EV_T // 128, z2, 0)
    plsc.subcore_barrier()

    def build_sidx(b, k):
        def bld(g, _):
            iv = idxF[b, pl.ds(g * 16, 16)]
            inb = jnp.right_shift(iv, CHB) == k
            li = jnp.where(inb, iv & (CH - 1),
                           CH + ((iv + lane) & (NTRASH - 1)))
            sidxB[0, pl.ds(g * 16, 16)] = li
            return 0
        lax.fori_loop(0, 8, bld, 0)

    def chunk_body(k, _):
        lo = smem[k]
        hi = smem[8 + k]
        b0 = jnp.right_shift(lo, 7)
        b1 = jnp.right_shift(hi + 127, 7)

        def sblk(b, _):
            build_sidx(b, k)
            pltpu.async_copy(con_hbm.at[evid.at[b]], conB, sem).wait()
            pltpu.sync_copy(conB, acc.at[sidxB.at[0]], add=True)
            return 0
        lax.fori_loop(b0, b1, sblk, 0)
        plsc.subcore_barrier()

        glo = jnp.maximum(lo, c * EV_W)
        ghi = jnp.minimum(hi, (c + 1) * EV_W)
        gb0 = jnp.right_shift(glo, 7)
        gb1 = jnp.right_shift(ghi + 127, 7)

        def gblk(b, _):
            def bld2(g, _):
                iv = idxF[b, pl.ds(g * 16, 16)]
                ev = evid[b, pl.ds(g * 16, 16)]
                pos = b * 128 + g * 16 + lane
                inr = (pos >= glo) & (pos < ghi)
                gidxB[0, pl.ds(g * 16, 16)] = iv & (CH - 1)
                widxB[0, pl.ds(g * 16, 16)] = iv
                goutB[0, pl.ds(g * 16, 16)] = jnp.where(
                    inr, ev, NB + ((ev + lane) & (NTRASH - 1)))
                return 0
            lax.fori_loop(0, 8, bld2, 0)
            cp_a = pltpu.async_copy(acc.at[gidxB.at[0]], abuf, sem)
            cp_w = pltpu.async_copy(w0_hbm.at[widxB.at[0]], wbuf, sem2)
            cp_a.wait()
            cp_w.wait()

            def comb(e, _):
                for j in range(4):
                    sl = pl.ds(j * 16, 16)
                    wbuf[e, sl] = wbuf[e, sl] + abuf[e, sl] * LRC
                return 0
            lax.fori_loop(0, 128, comb, 0)
            pltpu.async_copy(wbuf, g_hbm.at[goutB.at[0]], sem).wait()
            return 0
        lax.fori_loop(gb0, gb1, gblk, 0)
        plsc.subcore_barrier()

        def zblk(b, _):
            build_sidx(b, k)
            pltpu.sync_copy(zbuf, acc.at[sidxB.at[0]])
            return 0
        lax.fori_loop(b0, b1, zblk, 0)
        plsc.subcore_barrier()
        return 0

    lax.fori_loop(0, NCH, chunk_body, 0)


_sc_act = pl.kernel(
    _sc_act_body,
    out_type=jax.ShapeDtypeStruct((NB + NTRASH, CD), jnp.float32),
    mesh=_mesh,
    compiler_params=_params,
    scratch_types=[
        pltpu.VMEM((EV_T + 16,), _i32),
        pltpu.VMEM((8, 128), _i32),
        pltpu.VMEM((8, 128), _i32),
        pltpu.VMEM((128, CD), jnp.float32),
        pltpu.VMEM((128, CD), jnp.float32),
        pltpu.VMEM((128, CD), jnp.float32),
        pltpu.VMEM((128, CD), jnp.float32),
        pltpu.VMEM((1, 128), _i32),
        pltpu.VMEM((1, 128), _i32),
        pltpu.VMEM((1, 128), _i32),
        pltpu.VMEM((1, 128), _i32),
        pltpu.SMEM((16,), _i32),
        pltpu.VMEM_SHARED((CH + NTRASH, CD), jnp.float32),
        pltpu.SemaphoreType.DMA,
        pltpu.SemaphoreType.DMA,
    ],
)


def _tc_gate_body(g_ref, c_ref, p_ref, v_ref):
    g = g_ref[...]
    cc = c_ref[...]
    act = jnp.sum(g * cc, axis=-1, keepdims=True)
    scale = jnp.where(act > THETA, jnp.float32(LRC), jnp.float32(0.0))
    v_ref[...] = scale * p_ref[...]


def _tc_gate(g, concept, phon):
    return pl.pallas_call(
        _tc_gate_body,
        grid=(NB // 2048,),
        in_specs=[pl.BlockSpec((2048, CD), lambda i: (i, 0))] * 3,
        out_specs=pl.BlockSpec((2048, CD), lambda i: (i, 0)),
        out_shape=jax.ShapeDtypeStruct((NB, CD), jnp.float32),
    )(g, concept, phon)


def _sc_out_body(wlp_hbm, idx_hbm, v_hbm, out_hbm,
                 idxS, idxF, evid, vB, dstage, sidxB, smem, dacc, sem):
    c = lax.axis_index("c")
    s = lax.axis_index("s")
    ev0 = s * EV_T
    lane = lax.iota(_i32, 16)

    _sort_events(idx_hbm, ev0, idxS, idxF, evid, smem)
    plsc.subcore_barrier()

    def scatter_chunk(k):
        lo = smem[k]
        hi = smem[8 + k]
        b0 = jnp.right_shift(lo, 7)
        b1 = jnp.right_shift(hi + 127, 7)
        kbase = k * CH

        def sblk(b, _):
            def bld(g, _):
                iv = idxF[b, pl.ds(g * 16, 16)]
                loc = iv - kbase
                inb = (loc >= 0) & (loc < CH)
                li = jnp.where(inb, loc,
                               CH + ((iv + lane) & (NTRASH - 1)))
                sidxB[0, pl.ds(g * 16, 16)] = li
                return 0
            lax.fori_loop(0, 8, bld, 0)
            pltpu.async_copy(v_hbm.at[evid.at[b]], vB, sem).wait()
            pltpu.sync_copy(vB, dacc.at[sidxB.at[0]], add=True)
            return 0
        lax.fori_loop(b0, b1, sblk, 0)

    def full_chunk(kk, _):
        k = 2 * kk + c
        kbase = k * CH

        def di(i, _):
            pltpu.sync_copy(
                wlp_hbm.at[pl.ds(kbase + s * EV_T + i * 128, 128)], dstage)
            pltpu.sync_copy(dstage, dacc.at[pl.ds(s * EV_T + i * 128, 128)])
            return 0
        lax.fori_loop(0, 8, di, 0)
        plsc.subcore_barrier()
        scatter_chunk(k)
        plsc.subcore_barrier()

        def wo(i, _):
            pltpu.sync_copy(dacc.at[pl.ds(s * EV_T + i * 128, 128)], dstage)
            pltpu.sync_copy(
                dstage, out_hbm.at[pl.ds(kbase + s * EV_T + i * 128, 128)])
            return 0
        lax.fori_loop(0, 8, wo, 0)
        plsc.subcore_barrier()
        return 0

    lax.fori_loop(0, 3, full_chunk, 0)

    @pl.when(c == 0)
    def _():
        kbase = NFULL * CH
        rows_t = LAST // 16
        pltpu.sync_copy(wlp_hbm.at[pl.ds(kbase + s * rows_t, rows_t)],
                        dstage.at[pl.ds(0, rows_t)])
        pltpu.sync_copy(dstage.at[pl.ds(0, rows_t)],
                        dacc.at[pl.ds(s * rows_t, rows_t)])
        plsc.subcore_barrier()
        scatter_chunk(NFULL)
        plsc.subcore_barrier()
        pltpu.sync_copy(dacc.at[pl.ds(s * rows_t, rows_t)],
                        dstage.at[pl.ds(0, rows_t)])
        pltpu.sync_copy(dstage.at[pl.ds(0, rows_t)],
                        out_hbm.at[pl.ds(kbase + s * rows_t, rows_t)])


_sc_out = pl.kernel(
    _sc_out_body,
    out_type=jax.ShapeDtypeStruct((NL, CD), jnp.float32),
    mesh=_mesh,
    compiler_params=_params,
    scratch_types=[
        pltpu.VMEM((EV_T + 16,), _i32),
        pltpu.VMEM((8, 128), _i32),
        pltpu.VMEM((8, 128), _i32),
        pltpu.VMEM((128, CD), jnp.float32),
        pltpu.VMEM((128, CD), jnp.float32),
        pltpu.VMEM((1, 128), _i32),
        pltpu.SMEM((16,), _i32),
        pltpu.VMEM_SHARED((CH + NTRASH, CD), jnp.float32),
        pltpu.SemaphoreType.DMA,
    ],
)


def kernel(W_C_to_L, W_L_to_P, idx, concept, phon):
    idx = idx.astype(jnp.int32)
    gpad = _sc_act(W_C_to_L, idx, concept)
    v = _tc_gate(gpad[:NB], concept, phon)
    return _sc_out(W_L_to_P, idx, v)


# speedup vs baseline: 1.2827x; 1.0634x over previous
import jax
import jax.numpy as jnp
from jax import lax
from jax.experimental import pallas as pl
from jax.experimental.pallas import tpu as pltpu
from jax.experimental.pallas import tpu_sc as plsc

NL = 100000
CD = 64
NB = 16384
LRC = 0.05
THETA = 0.3

CH = 8192
CHB = 13
NTRASH = 128
NCH = 13
NFULL = NL // CH
LAST = NL - NFULL * CH
EV_T = NB // 16
ROWS_T = CH // 16
ACC_ROWS = CH + NTRASH

_mesh = plsc.VectorSubcoreMesh(core_axis_name="c", subcore_axis_name="s")
_params = pltpu.CompilerParams(use_tc_tiling_on_sc=False,
                               needs_layout_passes=False)

_i32 = jnp.int32
_f32 = jnp.float32


def _fused_body(w0_hbm, wlp_hbm, idx_hbm, con_hbm, phon_hbm, out_hbm,
                idxS, idxF, evid, conB, abuf, wbuf, phonB, zbuf,
                sidxB, gidxB, widxB, vidxB, smem,
                acc, dacc, sem, sem2, sem3, sem4):
    c = lax.axis_index("c")
    s = lax.axis_index("s")
    ev0 = s * EV_T
    lane = lax.iota(_i32, 16)

    pltpu.sync_copy(idx_hbm.at[pl.ds(ev0, EV_T)], idxS.at[pl.ds(0, EV_T)])

    def cntf(g, cnts):
        ck = jnp.right_shift(idxS[pl.ds(g * 16, 16)], CHB)
        return tuple(cnts[k] + jnp.sum(jnp.where(ck == k, 1, 0))
                     for k in range(NCH))
    cnts = lax.fori_loop(0, EV_T // 16, cntf, (_i32(0),) * NCH)

    running = _i32(0)
    for k in range(NCH):
        smem[k] = running
        smem[16 + k] = running
        running = running + cnts[k]

    lane0 = lane == 0

    def fill(e, _):
        iv = idxS[pl.ds(e, 16)][0]
        k = jnp.right_shift(iv, CHB)
        p = smem[16 + k]
        smem[16 + k] = p + 1
        r = jnp.full((16,), jnp.right_shift(p, 7), _i32)
        cc = jnp.full((16,), p & 127, _i32)
        plsc.store_scatter(idxF, [r, cc], jnp.full((16,), iv, _i32),
                           mask=lane0)
        plsc.store_scatter(evid, [r, cc], jnp.full((16,), ev0 + e, _i32),
                           mask=lane0)
        return 0
    lax.fori_loop(0, EV_T, fill, 0)

    zeros16 = jnp.zeros((16,), _f32)

    def zb(i, _):
        for j in range(4):
            zbuf[i, pl.ds(j * 16, 16)] = zeros16
        return 0
    lax.fori_loop(0, 128, zb, 0)

    for i in range(4):
        pltpu.sync_copy(zbuf, acc.at[pl.ds(s * ROWS_T + i * 128, 128)])

    @pl.when(s == 0)
    def _():
        pltpu.sync_copy(zbuf, acc.at[pl.ds(CH, NTRASH)])
    plsc.subcore_barrier()

    def build_sidx(b, k):
        def bld(g, _):
            iv = idxF[b, pl.ds(g * 16, 16)]
            inb = jnp.right_shift(iv, CHB) == k
            li = jnp.where(inb, iv & (CH - 1),
                           CH + ((iv + lane) & (NTRASH - 1)))
            sidxB[0, pl.ds(g * 16, 16)] = li
            return 0
        lax.fori_loop(0, 8, bld, 0)

    def do_chunk(k, rows_t):
        kbase = k * CH
        lo = smem[k]
        hi = smem[16 + k]
        b0 = jnp.right_shift(lo, 7)
        b1 = jnp.right_shift(hi + 127, 7)

        pltpu.sync_copy(wlp_hbm.at[pl.ds(kbase + s * rows_t, rows_t)],
                        dacc.at[pl.ds(s * rows_t, rows_t)])

        def sblk(b, _):
            cpc = pltpu.async_copy(con_hbm.at[evid.at[b]], conB, sem3)
            build_sidx(b, k)
            cpc.wait()
            pltpu.sync_copy(conB, acc.at[sidxB.at[0]], add=True)
            return 0
        lax.fori_loop(b0, b1, sblk, 0)
        plsc.subcore_barrier()

        def gblk(b, _):
            def bld2(g, _):
                iv = idxF[b, pl.ds(g * 16, 16)]
                loc = iv & (CH - 1)
                inb = jnp.right_shift(iv, CHB) == k
                gidxB[0, pl.ds(g * 16, 16)] = loc
                widxB[0, pl.ds(g * 16, 16)] = iv
                vidxB[0, pl.ds(g * 16, 16)] = jnp.where(
                    inb, loc, CH + ((iv + lane) & (NTRASH - 1)))
                return 0
            lax.fori_loop(0, 8, bld2, 0)
            cpa = pltpu.async_copy(acc.at[gidxB.at[0]], abuf, sem)
            cpw = pltpu.async_copy(w0_hbm.at[widxB.at[0]], wbuf, sem2)
            cpc = pltpu.async_copy(con_hbm.at[evid.at[b]], conB, sem3)
            cpp = pltpu.async_copy(phon_hbm.at[evid.at[b]], phonB, sem4)
            cpa.wait()
            cpw.wait()
            cpc.wait()
            cpp.wait()

            def dotf(e, _):
                sl = pl.ds(0, 16)
                r = ((wbuf[e, sl] + abuf[e, sl] * LRC) * conB[e, sl])
                for j in range(1, 4):
                    sl = pl.ds(j * 16, 16)
                    r = r + (wbuf[e, sl] + abuf[e, sl] * LRC) * conB[e, sl]
                act = jnp.sum(r)
                scale = jnp.where(act > THETA, _f32(LRC), _f32(0.0))
                for j in range(4):
                    sl = pl.ds(j * 16, 16)
                    phonB[e, sl] = phonB[e, sl] * scale
                return 0
            lax.fori_loop(0, 128, dotf, 0)
            pltpu.sync_copy(phonB, dacc.at[vidxB.at[0]], add=True)
            return 0
        lax.fori_loop(b0, b1, gblk, 0)
        plsc.subcore_barrier()

        pltpu.sync_copy(dacc.at[pl.ds(s * rows_t, rows_t)],
                        out_hbm.at[pl.ds(kbase + s * rows_t, rows_t)])

        def zblk(b, _):
            build_sidx(b, k)
            pltpu.sync_copy(zbuf, acc.at[sidxB.at[0]])
            return 0
        lax.fori_loop(b0, b1, zblk, 0)
        plsc.subcore_barrier()

    def full_chunk(kk, _):
        do_chunk(2 * kk + c, ROWS_T)
        return 0
    lax.fori_loop(0, NFULL // 2, full_chunk, 0)

    @pl.when(c == 0)
    def _():
        do_chunk(_i32(NFULL), LAST // 16)


_sc_fused = pl.kernel(
    _fused_body,
    out_type=jax.ShapeDtypeStruct((NL, CD), _f32),
    mesh=_mesh,
    compiler_params=_params,
    scratch_types=[
        pltpu.VMEM((EV_T + 16,), _i32),
        pltpu.VMEM((8, 128), _i32),
        pltpu.VMEM((8, 128), _i32),
        pltpu.VMEM((128, CD), _f32),
        pltpu.VMEM((128, CD), _f32),
        pltpu.VMEM((128, CD), _f32),
        pltpu.VMEM((128, CD), _f32),
        pltpu.VMEM((128, CD), _f32),
        pltpu.VMEM((1, 128), _i32),
        pltpu.VMEM((1, 128), _i32),
        pltpu.VMEM((1, 128), _i32),
        pltpu.VMEM((1, 128), _i32),
        pltpu.SMEM((32,), _i32),
        pltpu.VMEM_SHARED((ACC_ROWS, CD), _f32),
        pltpu.VMEM_SHARED((ACC_ROWS, CD), _f32),
        pltpu.SemaphoreType.DMA,
        pltpu.SemaphoreType.DMA,
        pltpu.SemaphoreType.DMA,
        pltpu.SemaphoreType.DMA,
    ],
)


def kernel(W_C_to_L, W_L_to_P, idx, concept, phon):
    return _sc_fused(W_C_to_L, W_L_to_P, idx.astype(_i32), concept, phon)
